# baseline (speedup 1.0000x reference)
import functools
import math

import jax
import jax.numpy as jnp
from jax import lax
from jax.experimental import pallas as pl
from jax.experimental.pallas import tpu as pltpu
from jax.experimental.pallas import tpu_sc as plsc

N_IN = 10000
N_FN = 50000
N_OUT = 10000
D = 128
H = 32
NH = 10
NG = 32

NCORE = 2
NSUB = 16
NW = NCORE * NSUB
SUB = 128
CHUNK = 128
KSUB = CHUNK // SUB
PADROWS = 128
ACC_F = 51200
ACC_IO = 10240
CNTW = 16
BLK = 2000

_INV_SQRT2 = 1.0 / math.sqrt(2.0)


def _gelu(x):
    return 0.5 * x * (1.0 + lax.erf(x * _INV_SQRT2))


IB = 10


def _round_edges(e):
    q = IB * NW * CHUNK
    return ((e + q - 1) // q) * q


def _sc_mesh():
    return plsc.VectorSubcoreMesh(core_axis_name="c", subcore_axis_name="s")


def _zero_acc(acc, zv, si, nrows):
    zper = nrows // NSUB // SUB

    def zb(i, c):
        pltpu.sync_copy(zv, acc.at[pl.ds((si * zper + i) * SUB, SUB)])
        return c

    lax.fori_loop(0, zper, zb, 0)


def _flush_acc(acc, out, ci, si, acc_rows):
    share = acc_rows // NSUB
    pltpu.sync_copy(acc.at[pl.ds(si * share, share)],
                    out.at[pl.ds(ci * acc_rows + si * share, share)])


def _agg_rel(P, s2d, d2d, out, n_dst, acc_rows, ci, si,
             acc, sidx, didx, rows, zv, sem0, sem1, ssem0, ssem1):
    _zero_acc(acc, zv, si, acc_rows)
    plsc.subcore_barrier()

    ew = (s2d.shape[0] * SUB) // NW
    nsb = ew // (CHUNK * IB)
    wrow = (ci * NSUB + si) * (ew // SUB)
    gsems = (sem0, sem1)
    ssems = (ssem0, ssem1)

    def load_idx(b, q):
        r0 = wrow + b * IB
        pltpu.sync_copy(s2d.at[pl.ds(r0, IB)], sidx.at[q])
        pltpu.sync_copy(d2d.at[pl.ds(r0, IB)], didx.at[q])

    def fire(q, i):
        p = i % 2
        pltpu.async_copy(P.at[sidx.at[q, i]], rows.at[p], gsems[p])

    def drain_fire_scatter(q, i):
        p = i % 2
        pltpu.make_async_copy(P.at[sidx.at[q, i]], rows.at[p],
                              gsems[p]).wait()
        pltpu.async_copy(rows.at[p], acc.at[didx.at[q, i]], ssems[p],
                         add=True)

    def wait_scatter(p):
        pltpu.make_async_copy(rows.at[p], acc.at[didx.at[0, 0]],
                              ssems[p]).wait()

    load_idx(0, 0)
    fire(0, 0)
    for i in range(1, IB):
        if i >= 2:
            wait_scatter(i % 2)
        fire(0, i)
        drain_fire_scatter(0, i - 1)

    def body(b, c):
        q = lax.rem(b, 2)
        load_idx(b, q)
        for i in range(IB):
            wait_scatter(i % 2)
            fire(q, i)
            if i == 0:
                drain_fire_scatter(1 - q, IB - 1)
            else:
                drain_fire_scatter(q, i - 1)
        return c

    lax.fori_loop(1, nsb, body, 0)
    drain_fire_scatter((nsb - 1) % 2, IB - 1)
    wait_scatter(0)
    wait_scatter(1)
    plsc.subcore_barrier()
    _flush_acc(acc, out, ci, si, acc_rows)
    plsc.subcore_barrier()


def _build_sc_layer(shapes_sd, last):
    rels = ["if", "ff", "of"] + ([] if last else ["fi", "fo"])
    ndst = {"if": N_FN, "ff": N_FN, "of": N_FN, "fi": N_IN, "fo": N_OUT}
    accr = {"if": ACC_F, "ff": ACC_F, "of": ACC_F, "fi": ACC_IO, "fo": ACC_IO}

    out_type = [jax.ShapeDtypeStruct((NCORE * accr[r], H), jnp.float32)
                for r in rels]

    @functools.partial(
        pl.kernel,
        out_type=out_type,
        mesh=_sc_mesh(),
        compiler_params=pltpu.CompilerParams(use_tc_tiling_on_sc=False),
        scratch_types=[
            pltpu.VMEM_SHARED((ACC_F, H), jnp.float32),
            pltpu.VMEM((2, IB, SUB), jnp.int32),
            pltpu.VMEM((2, IB, SUB), jnp.int32),
            pltpu.VMEM((2, SUB, H), jnp.float32),
            pltpu.VMEM((SUB, H), jnp.float32),
            pltpu.SemaphoreType.DMA,
            pltpu.SemaphoreType.DMA,
            pltpu.SemaphoreType.DMA,
            pltpu.SemaphoreType.DMA,
        ],
    )
    def k(*args):
        nr = len(rels)
        Ps = args[0:nr]
        s2ds = args[nr:2 * nr]
        d2ds = args[2 * nr:3 * nr]
        zeros_hbm = args[3 * nr]
        outs = args[3 * nr + 1:3 * nr + 1 + nr]
        (acc, sidx, didx, rows, zv,
         sem0, sem1, ssem0, ssem1) = args[3 * nr + 1 + nr:]
        ci = lax.axis_index("c")
        si = lax.axis_index("s")
        pltpu.sync_copy(zeros_hbm, zv)
        for t, r in enumerate(rels):
            _agg_rel(Ps[t], s2ds[t], d2ds[t], outs[t], ndst[r], accr[r],
                     ci, si, acc, sidx, didx, rows, zv,
                     sem0, sem1, ssem0, ssem1)

    return k, rels


def _build_sc_counts():
    rels = ["if", "ff", "of", "fi", "fo"]
    ndst = {"if": N_FN, "ff": N_FN, "of": N_FN, "fi": N_IN, "fo": N_OUT}
    accr = {"if": ACC_F, "ff": ACC_F, "of": ACC_F, "fi": ACC_IO, "fo": ACC_IO}

    out_type = [jax.ShapeDtypeStruct((NCORE * accr[r], CNTW), jnp.float32)
                for r in rels]

    @functools.partial(
        pl.kernel,
        out_type=out_type,
        mesh=_sc_mesh(),
        compiler_params=pltpu.CompilerParams(use_tc_tiling_on_sc=False),
        scratch_types=[
            pltpu.VMEM_SHARED((ACC_F, CNTW), jnp.float32),
            pltpu.VMEM((8, SUB), jnp.int32),
            pltpu.VMEM((SUB, CNTW), jnp.float32),
            pltpu.VMEM((SUB, CNTW), jnp.float32),
        ],
    )
    def k(*args):
        d2ds = args[0:5]
        zeros_hbm = args[5]
        ones_hbm = args[6]
        outs = args[7:12]
        acc, didx, zv, ones_v = args[12:]
        ci = lax.axis_index("c")
        si = lax.axis_index("s")
        pltpu.sync_copy(zeros_hbm, zv)
        pltpu.sync_copy(ones_hbm, ones_v)
        for t, r in enumerate(rels):
            _zero_acc(acc, zv, si, accr[r])
            plsc.subcore_barrier()
            d2d = d2ds[t]
            ew = (d2d.shape[0] * SUB) // NW
            nchunks = (ew // SUB) // 8
            wrow = (ci * NSUB + si) * (ew // SUB)

            def cb(kk, c, d2d=d2d, wrow=wrow):
                r0 = wrow + kk * 8
                pltpu.sync_copy(d2d.at[pl.ds(r0, 8)], didx)
                for j in range(8):
                    pltpu.sync_copy(ones_v, acc.at[didx.at[j]], add=True)
                return c

            lax.fori_loop(0, nchunks, cb, 0)
            plsc.subcore_barrier()
            _flush_acc(acc, outs[t], ci, si, accr[r])
            plsc.subcore_barrier()

    return k


def _dot(a, b):
    return jnp.dot(a, b, preferred_element_type=jnp.float32,
                   precision=lax.Precision.HIGHEST)


def _dot(a, b):
    return jnp.dot(a, b, preferred_element_type=jnp.float32,
                   precision=lax.Precision.HIGHEST)


def _ln_gelu(nf, lw_ref, lb_ref):
    g = _gelu(nf)
    mu = jnp.mean(g, axis=-1, keepdims=True)
    var = jnp.mean((g - mu) ** 2, axis=-1, keepdims=True)
    return (g - mu) / jnp.sqrt(var + 1e-5) * lw_ref[...] + lb_ref[...]


def _combine(aggs, cnts, xdst, mask2, WLs, WRs, BLs, lnw, lnb):
    n = xdst.shape[0]
    kd = xdst.shape[1]
    nrel = len(aggs)
    ncol = len(aggs[0])
    blk = 400 if ncol > 1 else BLK
    grid = n // blk

    def body(*refs):
        it = iter(refs)
        a_refs = [[next(it) for _ in range(ncol)] for _ in range(nrel)]
        c_refs = [next(it) for _ in range(nrel)]
        x_ref = next(it)
        m_ref = next(it) if mask2 is not None else None
        wl_refs = [next(it) for _ in range(nrel)]
        wr_refs = [next(it) for _ in range(nrel)]
        bl_refs = [next(it) for _ in range(nrel)]
        lw = next(it)
        lb = next(it)
        out = next(it)

        x = x_ref[...]
        nf = None
        for t in range(nrel):
            cnt = c_refs[t][0][:, 0:1] + c_refs[t][1][:, 0:1]
            inv = 1.0 / jnp.maximum(cnt, 1.0)
            mean = jnp.concatenate(
                [(a_refs[t][c][0] + a_refs[t][c][1]) * inv
                 for c in range(ncol)], axis=1)
            sage = (jnp.dot(mean, wl_refs[t][...],
                            preferred_element_type=jnp.float32)
                    + bl_refs[t][...]
                    + jnp.dot(x, wr_refs[t][...],
                              preferred_element_type=jnp.float32))
            nf = sage if nf is None else nf + sage
        if m_ref is not None:
            nf = nf * m_ref[...]
        out[...] = _ln_gelu(nf, lw, lb)

    a_spec = pl.BlockSpec((NCORE, blk, H), lambda i: (0, i, 0))
    c_spec = pl.BlockSpec((NCORE, blk, CNTW), lambda i: (0, i, 0))
    s_spec = pl.BlockSpec((1, H), lambda i: (0, 0))
    in_specs = []
    args = []
    for t in range(nrel):
        for c in range(ncol):
            in_specs.append(a_spec)
            args.append(aggs[t][c])
    for t in range(nrel):
        in_specs.append(c_spec)
        args.append(cnts[t])
    in_specs.append(pl.BlockSpec((blk, kd), lambda i: (i, 0)))
    args.append(xdst)
    if mask2 is not None:
        in_specs.append(pl.BlockSpec((blk, 1), lambda i: (i, 0)))
        args.append(mask2)
    for ws in (WLs, WRs):
        for w in ws:
            in_specs.append(pl.BlockSpec(w.shape, lambda i: (0, 0)))
            args.append(w)
    for b in BLs:
        in_specs.append(s_spec)
        args.append(b)
    in_specs += [s_spec, s_spec]
    args += [lnw, lnb]

    return pl.pallas_call(
        body,
        grid=(grid,),
        in_specs=in_specs,
        out_specs=pl.BlockSpec((blk, H), lambda i: (i, 0)),
        out_shape=jax.ShapeDtypeStruct((n, H), jnp.float32),
    )(*args)


def _pool(xf, mask2, batch2, att_w, lin_w, lin_b):
    nblk = N_FN // BLK

    def body(xf_ref, m_ref, b_ref, aw_ref, lw_ref, lb_ref, o_ref,
             smax_s, den_s, num_s):
        p = pl.program_id(0)
        i = pl.program_id(1)
        xfm = xf_ref[...] * m_ref[...]
        gid = lax.broadcasted_iota(jnp.int32, (BLK, NG), 1)
        oneh = (b_ref[...] == gid).astype(jnp.float32)
        scores = jnp.dot(xfm, aw_ref[...],
                         preferred_element_type=jnp.float32)
        neg = jnp.float32(-jnp.inf)

        @pl.when(p == 0)
        def _phase0():
            rows = []
            for g in range(NG):
                mg = jnp.where(oneh[:, g:g + 1] > 0.0, scores, neg)
                rows.append(jnp.max(mg, axis=0, keepdims=True))
            bm = jnp.concatenate(rows, axis=0)

            @pl.when(i == 0)
            def _():
                smax_s[...] = bm

            @pl.when(i > 0)
            def _():
                smax_s[...] = jnp.maximum(smax_s[...], bm)

        @pl.when(p == 1)
        def _phase1():
            smax = smax_s[...]
            smax = jnp.where(jnp.isfinite(smax), smax, 0.0)
            shift = _dot(oneh, smax)
            ex = jnp.exp(scores - shift)
            den = lax.dot_general(oneh, ex, (((0,), (0,)), ((), ())),
                                  preferred_element_type=jnp.float32,
                                  precision=lax.Precision.HIGHEST)
            nums = []
            for h in range(NH):
                wh = oneh * ex[:, h:h + 1]
                nums.append(lax.dot_general(
                    wh, xfm, (((0,), (0,)), ((), ())),
                    preferred_element_type=jnp.float32,
                    precision=lax.Precision.HIGHEST))
            num = jnp.concatenate(nums, axis=0)

            @pl.when(i == 0)
            def _():
                den_s[...] = den
                num_s[...] = num

            @pl.when(i > 0)
            def _():
                den_s[...] = den_s[...] + den
                num_s[...] = num_s[...] + num

        @pl.when((p == 1) & (i == nblk - 1))
        def _epilogue():
            den = jnp.maximum(den_s[...], 1e-9)
            acc = jnp.zeros((NG, 1), jnp.float32)
            for h in range(NH):
                ph = num_s[h * NG:(h + 1) * NG, :] / den[:, h:h + 1]
                acc = acc + jnp.dot(_gelu(ph),
                                    lw_ref[h * H:(h + 1) * H, :],
                                    preferred_element_type=jnp.float32)
            o_ref[...] = acc + lb_ref[...]

    return pl.pallas_call(
        body,
        grid=(2, nblk),
        in_specs=[pl.BlockSpec((BLK, H), lambda p, i: (i, 0)),
                  pl.BlockSpec((BLK, 1), lambda p, i: (i, 0)),
                  pl.BlockSpec((BLK, 1), lambda p, i: (i, 0)),
                  pl.BlockSpec(att_w.shape, lambda p, i: (0, 0)),
                  pl.BlockSpec(lin_w.shape, lambda p, i: (0, 0)),
                  pl.BlockSpec(lin_b.shape, lambda p, i: (0, 0))],
        out_specs=pl.BlockSpec((NG, 1), lambda p, i: (0, 0)),
        out_shape=jax.ShapeDtypeStruct((NG, 1), jnp.float32),
        scratch_shapes=[pltpu.VMEM((NG, NH), jnp.float32),
                        pltpu.VMEM((NG, NH), jnp.float32),
                        pltpu.VMEM((NH * NG, H), jnp.float32)],
    )(xf, mask2, batch2, att_w, lin_w, lin_b)


def kernel(x_input, x_function, x_output, edge_index_if, edge_index_fi,
           edge_index_ff, edge_index_of, edge_index_fo, batch, mask,
           Wl0, bl0, Wr0, Wl, bl, Wr, ln_w, ln_b, att_w, lin_w, lin_b):
    f32 = jnp.float32
    mask2 = mask[:, None].astype(f32)
    batch2 = batch[:, None].astype(jnp.int32)
    zeros32 = jnp.zeros((SUB, H), f32)
    zeros16 = jnp.zeros((SUB, CNTW), f32)
    ones16 = jnp.ones((SUB, CNTW), f32)

    def prep_edges(ei, n_src, n_dst):
        e = ei.shape[1]
        ep = _round_edges(e)
        pad = ep - e
        ar = jnp.arange(pad, dtype=jnp.int32)
        s = jnp.concatenate([ei[0].astype(jnp.int32), ar % n_src])
        dd = jnp.concatenate([ei[1].astype(jnp.int32),
                              n_dst + (ar % PADROWS)])
        return s.reshape(ep // SUB, SUB), dd.reshape(ep // SUB, SUB)

    sif, dif = prep_edges(edge_index_if, N_IN, N_FN)
    sff, dff = prep_edges(edge_index_ff, N_FN, N_FN)
    sof, dof = prep_edges(edge_index_of, N_OUT, N_FN)
    sfi, dfi = prep_edges(edge_index_fi, N_FN, N_IN)
    sfo, dfo = prep_edges(edge_index_fo, N_FN, N_OUT)

    counts_k = _build_sc_counts()
    cr = counts_k(dif, dff, dof, dfi, dfo, zeros16, ones16)
    c_if, c_ff, c_of, c_fi, c_fo = [
        c.reshape(NCORE, n, CNTW)
        for c, n in zip(cr, (ACC_F, ACC_F, ACC_F, ACC_IO, ACC_IO))]

    layer_full, _ = _build_sc_layer(None, last=False)
    layer_last, _ = _build_sc_layer(None, last=True)

    def run_full(xi_t, xf_t, xo_t):
        o = layer_full(xi_t, xf_t, xo_t, xf_t, xf_t,
                       sif, sff, sof, sfi, sfo,
                       dif, dff, dof, dfi, dfo, zeros32)
        return [x.reshape(NCORE, a, H) for x, a in
                zip(o, (ACC_F, ACC_F, ACC_F, ACC_IO, ACC_IO))]

    lnw2 = ln_w[None, :]
    lnb2 = ln_b[None, :]
    xf0 = x_function * mask2
    xi0 = x_input
    xo0 = x_output

    cols = [run_full(xi0[:, 32 * c:32 * (c + 1)],
                     xf0[:, 32 * c:32 * (c + 1)],
                     xo0[:, 32 * c:32 * (c + 1)]) for c in range(4)]
    aggs_f = [[cols[c][t] for c in range(4)] for t in range(3)]
    aggs_i = [[cols[c][3] for c in range(4)]]
    aggs_o = [[cols[c][4] for c in range(4)]]

    def wsel(l):
        if l == 0:
            return Wl0, bl0, Wr0
        return Wl[l - 1], bl[l - 1], Wr[l - 1]

    def combine_all(l, aggs_f, aggs_i, aggs_o, xf_t, xi_t, xo_t):
        WL, BL, WR = wsel(l)
        xf_n = _combine(aggs_f, (c_if, c_ff, c_of), xf_t, mask2,
                        [WL[0], WL[2], WL[3]], [WR[0], WR[2], WR[3]],
                        [BL[0][None, :], BL[2][None, :], BL[3][None, :]],
                        lnw2, lnb2)
        xi_n = _combine(aggs_i, (c_fi,), xi_t, None,
                        [WL[1]], [WR[1]], [BL[1][None, :]], lnw2, lnb2)
        xo_n = _combine(aggs_o, (c_fo,), xo_t, None,
                        [WL[4]], [WR[4]], [BL[4][None, :]], lnw2, lnb2)
        return xf_n, xi_n, xo_n

    xf_t, xi_t, xo_t = combine_all(0, aggs_f, aggs_i, aggs_o, xf0, xi0, xo0)

    for l in range(1, 5):
        WL, BL, WR = wsel(l)
        if l < 4:
            o = run_full(xi_t, xf_t, xo_t)
            aggs_f = [[o[0]], [o[1]], [o[2]]]
            xf_t, xi_t, xo_t = combine_all(l, aggs_f, [[o[3]]], [[o[4]]],
                                           xf_t, xi_t, xo_t)
        else:
            o = layer_last(xi_t, xf_t, xo_t, sif, sff, sof,
                           dif, dff, dof, zeros32)
            o = [x.reshape(NCORE, ACC_F, H) for x in o]
            xf_t = _combine([[o[0]], [o[1]], [o[2]]], (c_if, c_ff, c_of),
                            xf_t, mask2,
                            [WL[0], WL[2], WL[3]], [WR[0], WR[2], WR[3]],
                            [BL[0][None, :], BL[2][None, :], BL[3][None, :]],
                            lnw2, lnb2)

    return _pool(xf_t, mask2, batch2, att_w,
                 lin_w, lin_b[None, :].astype(f32))
